# Initial kernel scaffold; baseline (speedup 1.0000x reference)
import jax
import jax.numpy as jnp
from jax.experimental import pallas as pl


def kernel(x_t, s_t, t, h_t, edge_index, iproj_W, iproj_b, layers, oc_W, oc_b, of_W, of_b):
    raise NotImplementedError("write your pallas kernel here")


# speedup vs baseline: 4.3551x; 4.3551x over previous
import functools

import jax
import jax.numpy as jnp
import numpy as np
from jax import lax
from jax.experimental import pallas as pl
from jax.experimental.pallas import tpu as pltpu
from jax.experimental.pallas import tpu_sc as plsc

H = 64
F = 128
NC = 2
NS = 16
NW = NC * NS
IB = 128
BN = 512
BE = 512


def _silu(v):
    return v * jax.nn.sigmoid(v)


def _node_proj_body(nin_ref, w_ref, b_ref, o_ref):
    o_ref[...] = _silu(
        jnp.dot(nin_ref[...], w_ref[...], preferred_element_type=jnp.float32)
        + b_ref[...])


def _node_proj(node_in, w, b):
    n, k = node_in.shape
    grid = pl.cdiv(n, BN)
    return pl.pallas_call(
        _node_proj_body,
        grid=(grid,),
        in_specs=[
            pl.BlockSpec((BN, k), lambda i: (i, 0)),
            pl.BlockSpec((k, H), lambda i: (0, 0)),
            pl.BlockSpec((1, H), lambda i: (0, 0)),
        ],
        out_specs=pl.BlockSpec((BN, H), lambda i: (i, 0)),
        out_shape=jax.ShapeDtypeStruct((n, H), jnp.float32),
    )(node_in, w, b)


def _tables_body(x_ref, pos_ref, s_ref, w_ref, b_ref, d_ref, s_out_ref):
    x = x_ref[...]
    pos = pos_ref[...]
    wd = w_ref[0:H, :]
    ws = w_ref[H:2 * H, :]
    xd = jnp.dot(x, wd, preferred_element_type=jnp.float32) + b_ref[...]
    xs = jnp.dot(x, ws, preferred_element_type=jnp.float32)
    rows = x.shape[0]
    zpad_d = jnp.zeros((rows, F - H - 3), jnp.float32)
    zpad_s = jnp.zeros((rows, F - H - 4), jnp.float32)
    d_ref[...] = jnp.concatenate([xd, pos, zpad_d], axis=1)
    s_out_ref[...] = jnp.concatenate([xs, pos, s_ref[...], zpad_s], axis=1)


def _build_tables(x, pos, s2, ew1, eb1):
    n = x.shape[0]
    grid = pl.cdiv(n, BN)
    return pl.pallas_call(
        _tables_body,
        grid=(grid,),
        in_specs=[
            pl.BlockSpec((BN, H), lambda i: (i, 0)),
            pl.BlockSpec((BN, 3), lambda i: (i, 0)),
            pl.BlockSpec((BN, 1), lambda i: (i, 0)),
            pl.BlockSpec((2 * H + 1, H), lambda i: (0, 0)),
            pl.BlockSpec((1, H), lambda i: (0, 0)),
        ],
        out_specs=[
            pl.BlockSpec((BN, F), lambda i: (i, 0)),
            pl.BlockSpec((BN, F), lambda i: (i, 0)),
        ],
        out_shape=[
            jax.ShapeDtypeStruct((n, F), jnp.float32),
            jax.ShapeDtypeStruct((n, F), jnp.float32),
        ],
    )(x, pos, s2, ew1, eb1)


def _edge_mlp_body(d_ref, s_ref, w2_ref, b2_ref, cw_ref, cb_ref, wr_ref, o_ref):
    d = d_ref[...]
    s = s_ref[...]
    diff = d[:, H:H + 3] - s[:, H:H + 3]
    r2 = jnp.sum(diff * diff, axis=1, keepdims=True)
    e = d[:, :H] + s[:, :H] + r2 * wr_ref[...]
    h1 = _silu(e)
    m = _silu(
        jnp.dot(h1, w2_ref[...], preferred_element_type=jnp.float32)
        + b2_ref[...])
    m = m * s[:, H + 3:H + 4]
    gamma = jnp.sum(m * cw_ref[...], axis=1, keepdims=True) + cb_ref[...]
    rows = m.shape[0]
    ones = jnp.ones((rows, 1), jnp.float32)
    zpad = jnp.zeros((rows, F - H - 4), jnp.float32)
    o_ref[...] = jnp.concatenate([m, gamma * diff, ones, zpad], axis=1)


def _edge_mlp(dg, sg, ew2, eb2, cw, cb, wr):
    e = dg.shape[0]
    grid = e // BE
    return pl.pallas_call(
        _edge_mlp_body,
        grid=(grid,),
        in_specs=[
            pl.BlockSpec((BE, F), lambda i: (i, 0)),
            pl.BlockSpec((BE, F), lambda i: (i, 0)),
            pl.BlockSpec((H, H), lambda i: (0, 0)),
            pl.BlockSpec((1, H), lambda i: (0, 0)),
            pl.BlockSpec((1, H), lambda i: (0, 0)),
            pl.BlockSpec((1, 1), lambda i: (0, 0)),
            pl.BlockSpec((1, H), lambda i: (0, 0)),
        ],
        out_specs=pl.BlockSpec((BE, F), lambda i: (i, 0)),
        out_shape=jax.ShapeDtypeStruct((e, F), jnp.float32),
    )(dg, sg, ew2, eb2, cw, cb, wr)


def _node_update_body(p0_ref, p1_ref, x_ref, pos_ref, w1_ref, b1_ref,
                      w2_ref, b2_ref, xo_ref, poso_ref):
    acc = p0_ref[...] + p1_ref[...]
    msum = acc[:, :H]
    cu = acc[:, H:H + 3]
    deg = acc[:, H + 3:H + 4]
    degc = jnp.maximum(deg, 1.0)
    mavg = msum / degc
    x = x_ref[...]
    w1a = w1_ref[0:H, :]
    w1b = w1_ref[H:2 * H, :]
    h = _silu(
        jnp.dot(x, w1a, preferred_element_type=jnp.float32)
        + jnp.dot(mavg, w1b, preferred_element_type=jnp.float32)
        + b1_ref[...])
    xo_ref[...] = (
        jnp.dot(h, w2_ref[...], preferred_element_type=jnp.float32)
        + b2_ref[...])
    poso_ref[...] = pos_ref[...] + cu / degc


def _node_update(p0, p1, x, pos, nw1, nb1, nw2, nb2):
    n = x.shape[0]
    grid = pl.cdiv(n, BN)
    return pl.pallas_call(
        _node_update_body,
        grid=(grid,),
        in_specs=[
            pl.BlockSpec((BN, F), lambda i: (i, 0)),
            pl.BlockSpec((BN, F), lambda i: (i, 0)),
            pl.BlockSpec((BN, H), lambda i: (i, 0)),
            pl.BlockSpec((BN, 3), lambda i: (i, 0)),
            pl.BlockSpec((2 * H, H), lambda i: (0, 0)),
            pl.BlockSpec((1, H), lambda i: (0, 0)),
            pl.BlockSpec((H, H), lambda i: (0, 0)),
            pl.BlockSpec((1, H), lambda i: (0, 0)),
        ],
        out_specs=[
            pl.BlockSpec((BN, H), lambda i: (i, 0)),
            pl.BlockSpec((BN, 3), lambda i: (i, 0)),
        ],
        out_shape=[
            jax.ShapeDtypeStruct((n, H), jnp.float32),
            jax.ShapeDtypeStruct((n, 3), jnp.float32),
        ],
    )(p0, p1, x, pos, nw1, nb1, nw2, nb2)


def _out_proj_body(x_ref, wc_ref, bc_ref, wf_ref, bf_ref, xo_ref, ho_ref):
    x = x_ref[...]
    xo_ref[...] = (
        jnp.dot(x, wc_ref[...], preferred_element_type=jnp.float32)
        + bc_ref[...])
    ho_ref[...] = (
        jnp.dot(x, wf_ref[...], preferred_element_type=jnp.float32)
        + bf_ref[...])


def _out_proj(x, oc_w, oc_b, of_w, of_b):
    n = x.shape[0]
    nd = of_w.shape[1]
    grid = pl.cdiv(n, BN)
    return pl.pallas_call(
        _out_proj_body,
        grid=(grid,),
        in_specs=[
            pl.BlockSpec((BN, H), lambda i: (i, 0)),
            pl.BlockSpec((H, 3), lambda i: (0, 0)),
            pl.BlockSpec((1, 3), lambda i: (0, 0)),
            pl.BlockSpec((H, nd), lambda i: (0, 0)),
            pl.BlockSpec((1, nd), lambda i: (0, 0)),
        ],
        out_specs=[
            pl.BlockSpec((BN, 3), lambda i: (i, 0)),
            pl.BlockSpec((BN, nd), lambda i: (i, 0)),
        ],
        out_shape=[
            jax.ShapeDtypeStruct((n, 3), jnp.float32),
            jax.ShapeDtypeStruct((n, nd), jnp.float32),
        ],
    )(x, oc_w, oc_b, of_w, of_b)


def _sc_mesh():
    return plsc.VectorSubcoreMesh(
        core_axis_name="c", subcore_axis_name="s",
        num_cores=NC, num_subcores=NS)


def _edge_split(num_rows):
    rpw = num_rows // NW
    tail = num_rows - rpw * NW
    group = 1
    for g in (4, 3, 2):
        if rpw % g == 0:
            group = g
            break
    return rpw, group, rpw // group, tail


def _sc_gather(dtab, stab, dst2, src2):
    n = dtab.shape[0]
    rows = dst2.shape[0]
    e = rows * IB
    rpw, grp, ngrp, tail = _edge_split(rows)
    ge = grp * IB

    @functools.partial(
        pl.kernel,
        out_type=[
            jax.ShapeDtypeStruct((e, F), jnp.float32),
            jax.ShapeDtypeStruct((e, F), jnp.float32),
        ],
        mesh=_sc_mesh(),
        scratch_types=[
            pltpu.VMEM((grp, 1, IB), jnp.int32),
            pltpu.VMEM((grp, 1, IB), jnp.int32),
            pltpu.VMEM((ge, F), jnp.float32),
            pltpu.VMEM((ge, F), jnp.float32),
            pltpu.SemaphoreType.DMA,
        ],
    )
    def body(dtab_h, stab_h, dst_h, src_h, dg_h, sg_h,
             idxd, idxs, dbuf, sbuf, sem):
        wid = lax.axis_index("s") * NC + lax.axis_index("c")
        row0 = wid * rpw

        def do_rows(r0, nrow):
            pltpu.sync_copy(dst_h.at[pl.ds(r0, nrow)], idxd.at[pl.ds(0, nrow)])
            pltpu.sync_copy(src_h.at[pl.ds(r0, nrow)], idxs.at[pl.ds(0, nrow)])
            cps = []
            for j in range(nrow):
                cps.append(pltpu.async_copy(
                    dtab_h.at[idxd.at[j, 0]], dbuf.at[pl.ds(j * IB, IB)], sem))
                cps.append(pltpu.async_copy(
                    stab_h.at[idxs.at[j, 0]], sbuf.at[pl.ds(j * IB, IB)], sem))
            for cp in cps:
                cp.wait()
            pltpu.sync_copy(dbuf.at[pl.ds(0, nrow * IB)],
                            dg_h.at[pl.ds(r0 * IB, nrow * IB)])
            pltpu.sync_copy(sbuf.at[pl.ds(0, nrow * IB)],
                            sg_h.at[pl.ds(r0 * IB, nrow * IB)])

        def group_step(g, carry):
            do_rows(row0 + g * grp, grp)
            return carry

        lax.fori_loop(0, ngrp, group_step, 0)

        if tail:
            @pl.when(wid < tail)
            def _():
                do_rows(rpw * NW + wid, 1)

    return body(dtab, stab, dst2, src2)


def _sc_scatter(m, dst2, zrows):
    n = zrows.shape[0]
    rows = dst2.shape[0]
    rpw, grp, ngrp, tail = _edge_split(rows)
    ge = grp * IB

    @functools.partial(
        pl.kernel,
        out_type=[
            jax.ShapeDtypeStruct((n, F), jnp.float32),
            jax.ShapeDtypeStruct((n, F), jnp.float32),
        ],
        mesh=_sc_mesh(),
        scratch_types=[
            pltpu.VMEM((grp, 1, IB), jnp.int32),
            pltpu.VMEM((ge, F), jnp.float32),
            pltpu.VMEM_SHARED((n, F), jnp.float32),
        ],
    )
    def body(m_h, dst_h, z_h, p0_h, p1_h, idxd, mbuf, acc):
        cid = lax.axis_index("c")
        sid = lax.axis_index("s")
        wid = sid * NC + cid
        row0 = wid * rpw

        @pl.when(sid == 0)
        def _():
            pltpu.sync_copy(z_h, acc)

        plsc.subcore_barrier()

        def do_rows(r0, nrow):
            pltpu.sync_copy(dst_h.at[pl.ds(r0, nrow)], idxd.at[pl.ds(0, nrow)])
            pltpu.sync_copy(m_h.at[pl.ds(r0 * IB, nrow * IB)],
                            mbuf.at[pl.ds(0, nrow * IB)])
            for j in range(nrow):
                pltpu.sync_copy(mbuf.at[pl.ds(j * IB, IB)],
                                acc.at[idxd.at[j, 0]], add=True)

        def group_step(g, carry):
            do_rows(row0 + g * grp, grp)
            return carry

        lax.fori_loop(0, ngrp, group_step, 0)

        if tail:
            @pl.when(wid < tail)
            def _():
                do_rows(rpw * NW + wid, 1)

        plsc.subcore_barrier()

        npw = (n // NS) // 8 * 8
        nrem = n - npw * NS

        @pl.when(cid == 0)
        def _():
            pltpu.sync_copy(acc.at[pl.ds(sid * npw, npw)],
                            p0_h.at[pl.ds(sid * npw, npw)])
            if nrem:
                @pl.when(sid == 0)
                def _():
                    pltpu.sync_copy(acc.at[pl.ds(npw * NS, nrem)],
                                    p0_h.at[pl.ds(npw * NS, nrem)])

        @pl.when(cid == 1)
        def _():
            pltpu.sync_copy(acc.at[pl.ds(sid * npw, npw)],
                            p1_h.at[pl.ds(sid * npw, npw)])
            if nrem:
                @pl.when(sid == 0)
                def _():
                    pltpu.sync_copy(acc.at[pl.ds(npw * NS, nrem)],
                                    p1_h.at[pl.ds(npw * NS, nrem)])

    return body(m, dst2, zrows)


def kernel(x_t, s_t, t, h_t, edge_index, iproj_W, iproj_b, layers,
           oc_W, oc_b, of_W, of_b):
    n = h_t.shape[0]
    e = edge_index.shape[1]
    time_dim = 16
    half = time_dim // 2

    freqs = jnp.exp(jnp.linspace(0.0, 1.0, half) * -4.0)
    angles = t[0] * freqs
    emb = jnp.concatenate([jnp.sin(angles), jnp.cos(angles)], axis=-1)
    t_emb = jnp.broadcast_to(emb[None, :], (n, time_dim))
    node_in = jnp.concatenate([h_t, s_t[:, None], t_emb], axis=-1)

    x = _node_proj(node_in, iproj_W, iproj_b.reshape(1, H))
    pos = x_t
    s2 = s_t[:, None]
    src2 = edge_index[0].reshape(e // IB, 1, IB)
    dst2 = edge_index[1].reshape(e // IB, 1, IB)
    zrows = jnp.zeros((n, F), jnp.float32)

    for p in layers:
        dtab, stab = _build_tables(x, pos, s2, p['eW1'],
                                   p['eb1'].reshape(1, H))
        dg, sg = _sc_gather(dtab, stab, dst2, src2)
        m = _edge_mlp(dg, sg, p['eW2'], p['eb2'].reshape(1, H),
                      p['cW'].reshape(1, H), p['cb'].reshape(1, 1),
                      p['eW1'][2 * H].reshape(1, H))
        p0, p1 = _sc_scatter(m, dst2, zrows)
        x, pos = _node_update(p0, p1, x, pos, p['nW1'],
                              p['nb1'].reshape(1, H), p['nW2'],
                              p['nb2'].reshape(1, H))

    x0_pred, h0_pred = _out_proj(x, oc_W, oc_b.reshape(1, 3),
                                 of_W, of_b.reshape(1, of_W.shape[1]))
    return (x0_pred, h0_pred)


# speedup vs baseline: 6.4208x; 1.4743x over previous
import functools

import jax
import jax.numpy as jnp
import numpy as np
from jax import lax
from jax.experimental import pallas as pl
from jax.experimental.pallas import tpu as pltpu
from jax.experimental.pallas import tpu_sc as plsc

H = 64
F = 128
NC = 2
NS = 16
NW = NC * NS
IB = 128
BN = 512


def _silu(v):
    return v * jax.nn.sigmoid(v)


def _masks():
    io = lax.broadcasted_iota(jnp.int32, (1, F), 1)
    auxm = jnp.where((io >= H) & (io < H + 3), 1.0, 0.0)
    oneh = jnp.where(io == H + 3, 1.0, 0.0)
    return auxm, oneh


def _edge_block(e):
    for b in (2560, 1280, 640, 512, 256, 128):
        if e % b == 0:
            return b
    return IB


def _tables(x, pos, s2, wd, ws, b1, auxsign_note=None):
    auxm, oneh = _masks()
    xd = jnp.dot(x, wd, preferred_element_type=jnp.float32) + b1
    xs = jnp.dot(x, ws, preferred_element_type=jnp.float32)
    dtab = xd + pos * auxm
    stab = xs - pos * auxm + s2 * oneh
    return dtab, stab


def _proj_body(nin_ref, w_ref, b_ref, pos_ref, s_ref, wd_ref, ws_ref, b1_ref,
               x_ref, d_ref, st_ref):
    x = _silu(
        jnp.dot(nin_ref[...], w_ref[...], preferred_element_type=jnp.float32)
        + b_ref[...])
    x_ref[...] = x
    d, st = _tables(x, pos_ref[...], s_ref[...], wd_ref[...], ws_ref[...],
                    b1_ref[...])
    d_ref[...] = d
    st_ref[...] = st


def _proj_and_tables(node_in, w, b, pos, s2, wd, ws, b1):
    n, k = node_in.shape
    grid = pl.cdiv(n, BN)
    return pl.pallas_call(
        _proj_body,
        grid=(grid,),
        in_specs=[
            pl.BlockSpec((BN, k), lambda i: (i, 0)),
            pl.BlockSpec((k, H), lambda i: (0, 0)),
            pl.BlockSpec((1, H), lambda i: (0, 0)),
            pl.BlockSpec((BN, F), lambda i: (i, 0)),
            pl.BlockSpec((BN, 1), lambda i: (i, 0)),
            pl.BlockSpec((H, F), lambda i: (0, 0)),
            pl.BlockSpec((H, F), lambda i: (0, 0)),
            pl.BlockSpec((1, F), lambda i: (0, 0)),
        ],
        out_specs=[
            pl.BlockSpec((BN, H), lambda i: (i, 0)),
            pl.BlockSpec((BN, F), lambda i: (i, 0)),
            pl.BlockSpec((BN, F), lambda i: (i, 0)),
        ],
        out_shape=[
            jax.ShapeDtypeStruct((n, H), jnp.float32),
            jax.ShapeDtypeStruct((n, F), jnp.float32),
            jax.ShapeDtypeStruct((n, F), jnp.float32),
        ],
    )(node_in, w, b, pos, s2, wd, ws, b1)


def _edge_mlp_body(d_ref, s_ref, w2_ref, b2_ref, cw_ref, cb_ref, wr_ref,
                   o_ref):
    auxm, oneh = _masks()
    ds = d_ref[...] + s_ref[...]
    aux = ds * auxm
    r2 = jnp.sum(aux * aux, axis=1, keepdims=True)
    h1 = _silu(ds + r2 * wr_ref[...])
    m = _silu(
        jnp.dot(h1, w2_ref[...], preferred_element_type=jnp.float32)
        + b2_ref[...])
    scol = jnp.sum(ds * oneh, axis=1, keepdims=True)
    m = m * scol
    gamma = jnp.sum(m * cw_ref[...], axis=1, keepdims=True) + cb_ref[...]
    o_ref[...] = m + gamma * aux + oneh


def _edge_mlp(dg, sg, w2p, b2p, cw, cb, wr):
    e = dg.shape[0]
    be = _edge_block(e)
    grid = e // be
    return pl.pallas_call(
        _edge_mlp_body,
        grid=(grid,),
        in_specs=[
            pl.BlockSpec((be, F), lambda i: (i, 0)),
            pl.BlockSpec((be, F), lambda i: (i, 0)),
            pl.BlockSpec((F, F), lambda i: (0, 0)),
            pl.BlockSpec((1, F), lambda i: (0, 0)),
            pl.BlockSpec((1, F), lambda i: (0, 0)),
            pl.BlockSpec((1, 1), lambda i: (0, 0)),
            pl.BlockSpec((1, F), lambda i: (0, 0)),
        ],
        out_specs=pl.BlockSpec((be, F), lambda i: (i, 0)),
        out_shape=jax.ShapeDtypeStruct((e, F), jnp.float32),
    )(dg, sg, w2p, b2p, cw, cb, wr)


def _node_core(p0, p1, x, w1a, w1bp, b1, w2, b2):
    auxm, oneh = _masks()
    acc = p0 + p1
    deg = jnp.sum(acc * oneh, axis=1, keepdims=True)
    degc = jnp.maximum(deg, 1.0)
    acn = acc / degc
    h = _silu(
        jnp.dot(x, w1a, preferred_element_type=jnp.float32)
        + jnp.dot(acn, w1bp, preferred_element_type=jnp.float32)
        + b1)
    xn = jnp.dot(h, w2, preferred_element_type=jnp.float32) + b2
    return xn, acc * auxm / degc


def _nu_body(p0_ref, p1_ref, x_ref, pos_ref, s_ref, w1a_ref, w1b_ref, b1_ref,
             w2_ref, b2_ref, wd_ref, ws_ref, b1e_ref,
             xo_ref, poso_ref, d_ref, st_ref):
    auxm, _ = _masks()
    xn, cupd = _node_core(p0_ref[...], p1_ref[...], x_ref[...], w1a_ref[...],
                          w1b_ref[...], b1_ref[...], w2_ref[...], b2_ref[...])
    posn = pos_ref[...] + cupd
    xo_ref[...] = xn
    poso_ref[...] = posn
    d, st = _tables(xn, posn, s_ref[...], wd_ref[...], ws_ref[...],
                    b1e_ref[...])
    d_ref[...] = d
    st_ref[...] = st


def _node_update(p0, p1, x, pos, s2, w1a, w1bp, b1, w2, b2, wd, ws, b1e):
    n = x.shape[0]
    grid = pl.cdiv(n, BN)
    return pl.pallas_call(
        _nu_body,
        grid=(grid,),
        in_specs=[
            pl.BlockSpec((BN, F), lambda i: (i, 0)),
            pl.BlockSpec((BN, F), lambda i: (i, 0)),
            pl.BlockSpec((BN, H), lambda i: (i, 0)),
            pl.BlockSpec((BN, F), lambda i: (i, 0)),
            pl.BlockSpec((BN, 1), lambda i: (i, 0)),
            pl.BlockSpec((H, H), lambda i: (0, 0)),
            pl.BlockSpec((F, H), lambda i: (0, 0)),
            pl.BlockSpec((1, H), lambda i: (0, 0)),
            pl.BlockSpec((H, H), lambda i: (0, 0)),
            pl.BlockSpec((1, H), lambda i: (0, 0)),
            pl.BlockSpec((H, F), lambda i: (0, 0)),
            pl.BlockSpec((H, F), lambda i: (0, 0)),
            pl.BlockSpec((1, F), lambda i: (0, 0)),
        ],
        out_specs=[
            pl.BlockSpec((BN, H), lambda i: (i, 0)),
            pl.BlockSpec((BN, F), lambda i: (i, 0)),
            pl.BlockSpec((BN, F), lambda i: (i, 0)),
            pl.BlockSpec((BN, F), lambda i: (i, 0)),
        ],
        out_shape=[
            jax.ShapeDtypeStruct((n, H), jnp.float32),
            jax.ShapeDtypeStruct((n, F), jnp.float32),
            jax.ShapeDtypeStruct((n, F), jnp.float32),
            jax.ShapeDtypeStruct((n, F), jnp.float32),
        ],
    )(p0, p1, x, pos, s2, w1a, w1bp, b1, w2, b2, wd, ws, b1e)


def _final_body(p0_ref, p1_ref, x_ref, w1a_ref, w1b_ref, b1_ref, w2_ref,
                b2_ref, wc_ref, bc_ref, wf_ref, bf_ref, xo_ref, ho_ref):
    xn, _ = _node_core(p0_ref[...], p1_ref[...], x_ref[...], w1a_ref[...],
                       w1b_ref[...], b1_ref[...], w2_ref[...], b2_ref[...])
    xo_ref[...] = (
        jnp.dot(xn, wc_ref[...], preferred_element_type=jnp.float32)
        + bc_ref[...])
    ho_ref[...] = (
        jnp.dot(xn, wf_ref[...], preferred_element_type=jnp.float32)
        + bf_ref[...])


def _node_final(p0, p1, x, w1a, w1bp, b1, w2, b2, oc_w, oc_b, of_w, of_b):
    n = x.shape[0]
    nd = of_w.shape[1]
    grid = pl.cdiv(n, BN)
    return pl.pallas_call(
        _final_body,
        grid=(grid,),
        in_specs=[
            pl.BlockSpec((BN, F), lambda i: (i, 0)),
            pl.BlockSpec((BN, F), lambda i: (i, 0)),
            pl.BlockSpec((BN, H), lambda i: (i, 0)),
            pl.BlockSpec((H, H), lambda i: (0, 0)),
            pl.BlockSpec((F, H), lambda i: (0, 0)),
            pl.BlockSpec((1, H), lambda i: (0, 0)),
            pl.BlockSpec((H, H), lambda i: (0, 0)),
            pl.BlockSpec((1, H), lambda i: (0, 0)),
            pl.BlockSpec((H, 3), lambda i: (0, 0)),
            pl.BlockSpec((1, 3), lambda i: (0, 0)),
            pl.BlockSpec((H, nd), lambda i: (0, 0)),
            pl.BlockSpec((1, nd), lambda i: (0, 0)),
        ],
        out_specs=[
            pl.BlockSpec((BN, 3), lambda i: (i, 0)),
            pl.BlockSpec((BN, nd), lambda i: (i, 0)),
        ],
        out_shape=[
            jax.ShapeDtypeStruct((n, 3), jnp.float32),
            jax.ShapeDtypeStruct((n, nd), jnp.float32),
        ],
    )(p0, p1, x, w1a, w1bp, b1, w2, b2, oc_w, oc_b, of_w, of_b)


def _sc_mesh():
    return plsc.VectorSubcoreMesh(
        core_axis_name="c", subcore_axis_name="s",
        num_cores=NC, num_subcores=NS)


def _edge_split(num_rows):
    rpw = num_rows // NW
    tail = num_rows - rpw * NW
    group = 1
    for g in (4, 3, 2):
        if rpw % g == 0:
            group = g
            break
    return rpw, group, rpw // group, tail


def _sc_gather(dtab, stab, dst2, src2):
    rows = dst2.shape[0]
    e = rows * IB
    rpw, grp, ngrp, tail = _edge_split(rows)
    ge = grp * IB

    @functools.partial(
        pl.kernel,
        out_type=[
            jax.ShapeDtypeStruct((e, F), jnp.float32),
            jax.ShapeDtypeStruct((e, F), jnp.float32),
        ],
        mesh=_sc_mesh(),
        scratch_types=[
            pltpu.VMEM((grp, 1, IB), jnp.int32),
            pltpu.VMEM((grp, 1, IB), jnp.int32),
            pltpu.VMEM((ge, F), jnp.float32),
            pltpu.VMEM((ge, F), jnp.float32),
            pltpu.SemaphoreType.DMA,
        ],
    )
    def body(dtab_h, stab_h, dst_h, src_h, dg_h, sg_h,
             idxd, idxs, dbuf, sbuf, sem):
        wid = lax.axis_index("s") * NC + lax.axis_index("c")
        row0 = wid * rpw

        def do_rows(r0, nrow):
            pltpu.sync_copy(dst_h.at[pl.ds(r0, nrow)], idxd.at[pl.ds(0, nrow)])
            pltpu.sync_copy(src_h.at[pl.ds(r0, nrow)], idxs.at[pl.ds(0, nrow)])
            cps = []
            for j in range(nrow):
                cps.append(pltpu.async_copy(
                    dtab_h.at[idxd.at[j, 0]], dbuf.at[pl.ds(j * IB, IB)], sem))
                cps.append(pltpu.async_copy(
                    stab_h.at[idxs.at[j, 0]], sbuf.at[pl.ds(j * IB, IB)], sem))
            for cp in cps:
                cp.wait()
            pltpu.sync_copy(dbuf.at[pl.ds(0, nrow * IB)],
                            dg_h.at[pl.ds(r0 * IB, nrow * IB)])
            pltpu.sync_copy(sbuf.at[pl.ds(0, nrow * IB)],
                            sg_h.at[pl.ds(r0 * IB, nrow * IB)])

        def group_step(g, carry):
            do_rows(row0 + g * grp, grp)
            return carry

        lax.fori_loop(0, ngrp, group_step, 0)

        if tail:
            @pl.when(wid < tail)
            def _():
                do_rows(rpw * NW + wid, 1)

    return body(dtab, stab, dst2, src2)


def _sc_scatter(m, dst2, zrows):
    n = zrows.shape[0]
    rows = dst2.shape[0]
    rpw, grp, ngrp, tail = _edge_split(rows)
    ge = grp * IB

    @functools.partial(
        pl.kernel,
        out_type=[
            jax.ShapeDtypeStruct((n, F), jnp.float32),
            jax.ShapeDtypeStruct((n, F), jnp.float32),
        ],
        mesh=_sc_mesh(),
        scratch_types=[
            pltpu.VMEM((grp, 1, IB), jnp.int32),
            pltpu.VMEM((ge, F), jnp.float32),
            pltpu.VMEM_SHARED((n, F), jnp.float32),
        ],
    )
    def body(m_h, dst_h, z_h, p0_h, p1_h, idxd, mbuf, acc):
        cid = lax.axis_index("c")
        sid = lax.axis_index("s")
        wid = sid * NC + cid
        row0 = wid * rpw

        @pl.when(sid == 0)
        def _():
            pltpu.sync_copy(z_h, acc)

        plsc.subcore_barrier()

        def do_rows(r0, nrow):
            pltpu.sync_copy(dst_h.at[pl.ds(r0, nrow)], idxd.at[pl.ds(0, nrow)])
            pltpu.sync_copy(m_h.at[pl.ds(r0 * IB, nrow * IB)],
                            mbuf.at[pl.ds(0, nrow * IB)])
            for j in range(nrow):
                pltpu.sync_copy(mbuf.at[pl.ds(j * IB, IB)],
                                acc.at[idxd.at[j, 0]], add=True)

        def group_step(g, carry):
            do_rows(row0 + g * grp, grp)
            return carry

        lax.fori_loop(0, ngrp, group_step, 0)

        if tail:
            @pl.when(wid < tail)
            def _():
                do_rows(rpw * NW + wid, 1)

        plsc.subcore_barrier()

        npw = (n // NS) // 8 * 8
        nrem = n - npw * NS

        @pl.when(cid == 0)
        def _():
            pltpu.sync_copy(acc.at[pl.ds(sid * npw, npw)],
                            p0_h.at[pl.ds(sid * npw, npw)])
            if nrem:
                @pl.when(sid == 0)
                def _():
                    pltpu.sync_copy(acc.at[pl.ds(npw * NS, nrem)],
                                    p0_h.at[pl.ds(npw * NS, nrem)])

        @pl.when(cid == 1)
        def _():
            pltpu.sync_copy(acc.at[pl.ds(sid * npw, npw)],
                            p1_h.at[pl.ds(sid * npw, npw)])
            if nrem:
                @pl.when(sid == 0)
                def _():
                    pltpu.sync_copy(acc.at[pl.ds(npw * NS, nrem)],
                                    p1_h.at[pl.ds(npw * NS, nrem)])

    return body(m, dst2, zrows)


def _pad_w(w, rows, cols):
    return jnp.pad(w, ((0, rows - w.shape[0]), (0, cols - w.shape[1])))


def kernel(x_t, s_t, t, h_t, edge_index, iproj_W, iproj_b, layers,
           oc_W, oc_b, of_W, of_b):
    n = h_t.shape[0]
    e = edge_index.shape[1]
    time_dim = 16
    half = time_dim // 2

    freqs = jnp.exp(jnp.linspace(0.0, 1.0, half) * -4.0)
    angles = t[0] * freqs
    emb = jnp.concatenate([jnp.sin(angles), jnp.cos(angles)], axis=-1)
    t_emb = jnp.broadcast_to(emb[None, :], (n, time_dim))
    node_in = jnp.concatenate([h_t, s_t[:, None], t_emb], axis=-1)

    s2 = s_t[:, None]
    posfull = jnp.pad(x_t, ((0, 0), (H, F - H - 3)))
    src2 = edge_index[0].reshape(e // IB, 1, IB)
    dst2 = edge_index[1].reshape(e // IB, 1, IB)
    zrows = jnp.zeros((n, F), jnp.float32)

    pw = []
    for p in layers:
        pw.append(dict(
            wd=_pad_w(p['eW1'][:H], H, F),
            ws=_pad_w(p['eW1'][H:2 * H], H, F),
            wr=_pad_w(p['eW1'][2 * H].reshape(1, H), 1, F),
            b1e=_pad_w(p['eb1'].reshape(1, H), 1, F),
            w2p=_pad_w(p['eW2'], F, F),
            b2p=_pad_w(p['eb2'].reshape(1, H), 1, F),
            cw=_pad_w(p['cW'].reshape(1, H), 1, F),
            cb=p['cb'].reshape(1, 1),
            w1a=p['nW1'][:H],
            w1bp=_pad_w(p['nW1'][H:], F, H),
            b1=p['nb1'].reshape(1, H),
            w2=p['nW2'],
            b2=p['nb2'].reshape(1, H),
        ))

    x, dtab, stab = _proj_and_tables(
        node_in, iproj_W, iproj_b.reshape(1, H), posfull, s2,
        pw[0]['wd'], pw[0]['ws'], pw[0]['b1e'])
    pos = posfull

    for li, p in enumerate(pw):
        dg, sg = _sc_gather(dtab, stab, dst2, src2)
        m = _edge_mlp(dg, sg, p['w2p'], p['b2p'], p['cw'], p['cb'], p['wr'])
        p0, p1 = _sc_scatter(m, dst2, zrows)
        if li + 1 < len(pw):
            nxt = pw[li + 1]
            x, pos, dtab, stab = _node_update(
                p0, p1, x, pos, s2, p['w1a'], p['w1bp'], p['b1'], p['w2'],
                p['b2'], nxt['wd'], nxt['ws'], nxt['b1e'])
        else:
            x0_pred, h0_pred = _node_final(
                p0, p1, x, p['w1a'], p['w1bp'], p['b1'], p['w2'], p['b2'],
                oc_W, oc_b.reshape(1, 3), of_W,
                of_b.reshape(1, of_W.shape[1]))

    return (x0_pred, h0_pred)


# speedup vs baseline: 6.9088x; 1.0760x over previous
import functools

import jax
import jax.numpy as jnp
import numpy as np
from jax import lax
from jax.experimental import pallas as pl
from jax.experimental.pallas import tpu as pltpu
from jax.experimental.pallas import tpu_sc as plsc

H = 64
F = 128
NC = 2
NS = 16
NW = NC * NS
IB = 128
BN = 512


def _silu(v):
    return v * jax.nn.sigmoid(v)


def _masks():
    io = lax.broadcasted_iota(jnp.int32, (1, F), 1)
    auxm = jnp.where((io >= H) & (io < H + 3), 1.0, 0.0)
    oneh = jnp.where(io == H + 3, 1.0, 0.0)
    return auxm, oneh


def _edge_block(e):
    for b in (2560, 1280, 640, 512, 256, 128):
        if e % b == 0:
            return b
    return IB


def _tables(x, pos, s2, wd, ws, b1, auxsign_note=None):
    auxm, oneh = _masks()
    xd = jnp.dot(x, wd, preferred_element_type=jnp.float32) + b1
    xs = jnp.dot(x, ws, preferred_element_type=jnp.float32)
    dtab = xd + pos * auxm
    stab = xs - pos * auxm + s2 * oneh
    return dtab, stab


def _proj_body(nin_ref, w_ref, b_ref, pos_ref, s_ref, wd_ref, ws_ref, b1_ref,
               x_ref, d_ref, st_ref):
    x = _silu(
        jnp.dot(nin_ref[...], w_ref[...], preferred_element_type=jnp.float32)
        + b_ref[...])
    x_ref[...] = x
    d, st = _tables(x, pos_ref[...], s_ref[...], wd_ref[...], ws_ref[...],
                    b1_ref[...])
    d_ref[...] = d
    st_ref[...] = st


def _proj_and_tables(node_in, w, b, pos, s2, wd, ws, b1):
    n, k = node_in.shape
    grid = pl.cdiv(n, BN)
    return pl.pallas_call(
        _proj_body,
        grid=(grid,),
        in_specs=[
            pl.BlockSpec((BN, k), lambda i: (i, 0)),
            pl.BlockSpec((k, H), lambda i: (0, 0)),
            pl.BlockSpec((1, H), lambda i: (0, 0)),
            pl.BlockSpec((BN, F), lambda i: (i, 0)),
            pl.BlockSpec((BN, 1), lambda i: (i, 0)),
            pl.BlockSpec((H, F), lambda i: (0, 0)),
            pl.BlockSpec((H, F), lambda i: (0, 0)),
            pl.BlockSpec((1, F), lambda i: (0, 0)),
        ],
        out_specs=[
            pl.BlockSpec((BN, H), lambda i: (i, 0)),
            pl.BlockSpec((BN, F), lambda i: (i, 0)),
            pl.BlockSpec((BN, F), lambda i: (i, 0)),
        ],
        out_shape=[
            jax.ShapeDtypeStruct((n, H), jnp.float32),
            jax.ShapeDtypeStruct((n, F), jnp.float32),
            jax.ShapeDtypeStruct((n, F), jnp.float32),
        ],
    )(node_in, w, b, pos, s2, wd, ws, b1)


def _edge_mlp_body(d_ref, s_ref, w2_ref, b2_ref, cw_ref, cb_ref, wr_ref,
                   o_ref):
    auxm, oneh = _masks()
    ds = d_ref[...] + s_ref[...]
    aux = ds * auxm
    r2 = jnp.sum(aux * aux, axis=1, keepdims=True)
    h1 = _silu(ds + r2 * wr_ref[...])
    m = _silu(
        jnp.dot(h1, w2_ref[...], preferred_element_type=jnp.float32)
        + b2_ref[...])
    scol = jnp.sum(ds * oneh, axis=1, keepdims=True)
    m = m * scol
    gamma = jnp.sum(m * cw_ref[...], axis=1, keepdims=True) + cb_ref[...]
    o_ref[...] = m + gamma * aux + oneh


def _edge_mlp(dg, sg, w2p, b2p, cw, cb, wr):
    e = dg.shape[0]
    be = _edge_block(e)
    grid = e // be
    return pl.pallas_call(
        _edge_mlp_body,
        grid=(grid,),
        in_specs=[
            pl.BlockSpec((be, F), lambda i: (i, 0)),
            pl.BlockSpec((be, F), lambda i: (i, 0)),
            pl.BlockSpec((F, F), lambda i: (0, 0)),
            pl.BlockSpec((1, F), lambda i: (0, 0)),
            pl.BlockSpec((1, F), lambda i: (0, 0)),
            pl.BlockSpec((1, 1), lambda i: (0, 0)),
            pl.BlockSpec((1, F), lambda i: (0, 0)),
        ],
        out_specs=pl.BlockSpec((be, F), lambda i: (i, 0)),
        out_shape=jax.ShapeDtypeStruct((e, F), jnp.float32),
    )(dg, sg, w2p, b2p, cw, cb, wr)


def _node_core(p0, p1, x, w1a, w1bp, b1, w2, b2):
    auxm, oneh = _masks()
    acc = p0 + p1
    deg = jnp.sum(acc * oneh, axis=1, keepdims=True)
    degc = jnp.maximum(deg, 1.0)
    acn = acc / degc
    h = _silu(
        jnp.dot(x, w1a, preferred_element_type=jnp.float32)
        + jnp.dot(acn, w1bp, preferred_element_type=jnp.float32)
        + b1)
    xn = jnp.dot(h, w2, preferred_element_type=jnp.float32) + b2
    return xn, acc * auxm / degc


def _nu_body(p0_ref, p1_ref, x_ref, pos_ref, s_ref, w1a_ref, w1b_ref, b1_ref,
             w2_ref, b2_ref, wd_ref, ws_ref, b1e_ref,
             xo_ref, poso_ref, d_ref, st_ref):
    auxm, _ = _masks()
    xn, cupd = _node_core(p0_ref[...], p1_ref[...], x_ref[...], w1a_ref[...],
                          w1b_ref[...], b1_ref[...], w2_ref[...], b2_ref[...])
    posn = pos_ref[...] + cupd
    xo_ref[...] = xn
    poso_ref[...] = posn
    d, st = _tables(xn, posn, s_ref[...], wd_ref[...], ws_ref[...],
                    b1e_ref[...])
    d_ref[...] = d
    st_ref[...] = st


def _node_update(p0, p1, x, pos, s2, w1a, w1bp, b1, w2, b2, wd, ws, b1e):
    n = x.shape[0]
    grid = pl.cdiv(n, BN)
    return pl.pallas_call(
        _nu_body,
        grid=(grid,),
        in_specs=[
            pl.BlockSpec((BN, F), lambda i: (i, 0)),
            pl.BlockSpec((BN, F), lambda i: (i, 0)),
            pl.BlockSpec((BN, H), lambda i: (i, 0)),
            pl.BlockSpec((BN, F), lambda i: (i, 0)),
            pl.BlockSpec((BN, 1), lambda i: (i, 0)),
            pl.BlockSpec((H, H), lambda i: (0, 0)),
            pl.BlockSpec((F, H), lambda i: (0, 0)),
            pl.BlockSpec((1, H), lambda i: (0, 0)),
            pl.BlockSpec((H, H), lambda i: (0, 0)),
            pl.BlockSpec((1, H), lambda i: (0, 0)),
            pl.BlockSpec((H, F), lambda i: (0, 0)),
            pl.BlockSpec((H, F), lambda i: (0, 0)),
            pl.BlockSpec((1, F), lambda i: (0, 0)),
        ],
        out_specs=[
            pl.BlockSpec((BN, H), lambda i: (i, 0)),
            pl.BlockSpec((BN, F), lambda i: (i, 0)),
            pl.BlockSpec((BN, F), lambda i: (i, 0)),
            pl.BlockSpec((BN, F), lambda i: (i, 0)),
        ],
        out_shape=[
            jax.ShapeDtypeStruct((n, H), jnp.float32),
            jax.ShapeDtypeStruct((n, F), jnp.float32),
            jax.ShapeDtypeStruct((n, F), jnp.float32),
            jax.ShapeDtypeStruct((n, F), jnp.float32),
        ],
    )(p0, p1, x, pos, s2, w1a, w1bp, b1, w2, b2, wd, ws, b1e)


def _final_body(p0_ref, p1_ref, x_ref, w1a_ref, w1b_ref, b1_ref, w2_ref,
                b2_ref, wc_ref, bc_ref, wf_ref, bf_ref, xo_ref, ho_ref):
    xn, _ = _node_core(p0_ref[...], p1_ref[...], x_ref[...], w1a_ref[...],
                       w1b_ref[...], b1_ref[...], w2_ref[...], b2_ref[...])
    xo_ref[...] = (
        jnp.dot(xn, wc_ref[...], preferred_element_type=jnp.float32)
        + bc_ref[...])
    ho_ref[...] = (
        jnp.dot(xn, wf_ref[...], preferred_element_type=jnp.float32)
        + bf_ref[...])


def _node_final(p0, p1, x, w1a, w1bp, b1, w2, b2, oc_w, oc_b, of_w, of_b):
    n = x.shape[0]
    nd = of_w.shape[1]
    grid = pl.cdiv(n, BN)
    return pl.pallas_call(
        _final_body,
        grid=(grid,),
        in_specs=[
            pl.BlockSpec((BN, F), lambda i: (i, 0)),
            pl.BlockSpec((BN, F), lambda i: (i, 0)),
            pl.BlockSpec((BN, H), lambda i: (i, 0)),
            pl.BlockSpec((H, H), lambda i: (0, 0)),
            pl.BlockSpec((F, H), lambda i: (0, 0)),
            pl.BlockSpec((1, H), lambda i: (0, 0)),
            pl.BlockSpec((H, H), lambda i: (0, 0)),
            pl.BlockSpec((1, H), lambda i: (0, 0)),
            pl.BlockSpec((H, 3), lambda i: (0, 0)),
            pl.BlockSpec((1, 3), lambda i: (0, 0)),
            pl.BlockSpec((H, nd), lambda i: (0, 0)),
            pl.BlockSpec((1, nd), lambda i: (0, 0)),
        ],
        out_specs=[
            pl.BlockSpec((BN, 3), lambda i: (i, 0)),
            pl.BlockSpec((BN, nd), lambda i: (i, 0)),
        ],
        out_shape=[
            jax.ShapeDtypeStruct((n, 3), jnp.float32),
            jax.ShapeDtypeStruct((n, nd), jnp.float32),
        ],
    )(p0, p1, x, w1a, w1bp, b1, w2, b2, oc_w, oc_b, of_w, of_b)


def _sc_mesh():
    return plsc.VectorSubcoreMesh(
        core_axis_name="c", subcore_axis_name="s",
        num_cores=NC, num_subcores=NS)


def _fire(descs):
    for d in descs:
        d.start()


def _drain(descs):
    for d in descs:
        d.wait()


def _sc_gather(dtab, stab, dst2, src2):
    rows = dst2.shape[0]
    e = rows * IB
    rpw = rows // NW
    tail = rows - rpw * NW
    assert rpw % 2 == 0
    ng2 = rpw // 2

    @functools.partial(
        pl.kernel,
        out_type=[
            jax.ShapeDtypeStruct((e, F), jnp.float32),
            jax.ShapeDtypeStruct((e, F), jnp.float32),
        ],
        mesh=_sc_mesh(),
        scratch_types=[
            pltpu.VMEM((2, 1, IB), jnp.int32),
            pltpu.VMEM((2, 1, IB), jnp.int32),
            pltpu.VMEM((2 * IB, F), jnp.float32),
            pltpu.VMEM((2 * IB, F), jnp.float32),
            pltpu.SemaphoreType.DMA,
            pltpu.SemaphoreType.DMA,
            pltpu.SemaphoreType.DMA,
            pltpu.SemaphoreType.DMA,
        ],
    )
    def body(dtab_h, stab_h, dst_h, src_h, dg_h, sg_h,
             idxd, idxs, dbuf, sbuf, gsem0, gsem1, wsem0, wsem1):
        wid = lax.axis_index("s") * NC + lax.axis_index("c")
        row0 = wid * rpw
        gsems = (gsem0, gsem1)
        wsems = (wsem0, wsem1)

        def load_idx(r, slot):
            pltpu.sync_copy(dst_h.at[pl.ds(r, 1)], idxd.at[pl.ds(slot, 1)])
            pltpu.sync_copy(src_h.at[pl.ds(r, 1)], idxs.at[pl.ds(slot, 1)])

        def g_descs(slot):
            return (
                pltpu.make_async_copy(dtab_h.at[idxd.at[slot, 0]],
                                      dbuf.at[pl.ds(slot * IB, IB)],
                                      gsems[slot]),
                pltpu.make_async_copy(stab_h.at[idxs.at[slot, 0]],
                                      sbuf.at[pl.ds(slot * IB, IB)],
                                      gsems[slot]),
            )

        def w_descs(r, slot):
            return (
                pltpu.make_async_copy(dbuf.at[pl.ds(slot * IB, IB)],
                                      dg_h.at[pl.ds(r * IB, IB)],
                                      wsems[slot]),
                pltpu.make_async_copy(sbuf.at[pl.ds(slot * IB, IB)],
                                      sg_h.at[pl.ds(r * IB, IB)],
                                      wsems[slot]),
            )

        load_idx(row0, 0)
        _fire(g_descs(0))

        def pair(g2, carry):
            r = row0 + 2 * g2

            @pl.when(g2 >= 1)
            def _():
                _drain(w_descs(r, 1))
            load_idx(r + 1, 1)
            _fire(g_descs(1))
            _drain(g_descs(0))
            _fire(w_descs(r, 0))

            _drain(w_descs(r, 0))
            @pl.when(g2 < ng2 - 1)
            def _():
                load_idx(r + 2, 0)
                _fire(g_descs(0))
            _drain(g_descs(1))
            _fire(w_descs(r + 1, 1))
            return carry

        lax.fori_loop(0, ng2, pair, 0)
        _drain(w_descs(row0, 1))

        if tail:
            @pl.when(wid < tail)
            def _():
                r = rpw * NW + wid
                load_idx(r, 0)
                gd = g_descs(0)
                _fire(gd)
                _drain(gd)
                wd = w_descs(r, 0)
                _fire(wd)
                _drain(wd)

    return body(dtab, stab, dst2, src2)


def _sc_scatter(m, dst2, zrows):
    n = zrows.shape[0]
    rows = dst2.shape[0]
    rpw = rows // NW
    tail = rows - rpw * NW
    assert rpw % 2 == 0
    ng2 = rpw // 2

    @functools.partial(
        pl.kernel,
        out_type=[
            jax.ShapeDtypeStruct((n, F), jnp.float32),
            jax.ShapeDtypeStruct((n, F), jnp.float32),
        ],
        mesh=_sc_mesh(),
        scratch_types=[
            pltpu.VMEM((2, 1, IB), jnp.int32),
            pltpu.VMEM((2 * IB, F), jnp.float32),
            pltpu.VMEM_SHARED((n, F), jnp.float32),
            pltpu.SemaphoreType.DMA,
            pltpu.SemaphoreType.DMA,
        ],
    )
    def body(m_h, dst_h, z_h, p0_h, p1_h, idxd, mbuf, acc, msem0, msem1):
        cid = lax.axis_index("c")
        sid = lax.axis_index("s")
        wid = sid * NC + cid
        row0 = wid * rpw
        msems = (msem0, msem1)

        @pl.when(sid == 0)
        def _():
            pltpu.sync_copy(z_h, acc)

        plsc.subcore_barrier()

        def m_descs(r, slot):
            return (
                pltpu.make_async_copy(m_h.at[pl.ds(r * IB, IB)],
                                      mbuf.at[pl.ds(slot * IB, IB)],
                                      msems[slot]),
                pltpu.make_async_copy(dst_h.at[pl.ds(r, 1)],
                                      idxd.at[pl.ds(slot, 1)],
                                      msems[slot]),
            )

        def sadd(slot):
            pltpu.sync_copy(mbuf.at[pl.ds(slot * IB, IB)],
                            acc.at[idxd.at[slot, 0]], add=True)

        _fire(m_descs(row0, 0))

        def pair(g2, carry):
            r = row0 + 2 * g2
            _fire(m_descs(r + 1, 1))
            _drain(m_descs(r, 0))
            sadd(0)
            @pl.when(g2 < ng2 - 1)
            def _():
                _fire(m_descs(r + 2, 0))
            _drain(m_descs(r + 1, 1))
            sadd(1)
            return carry

        lax.fori_loop(0, ng2, pair, 0)

        if tail:
            @pl.when(wid < tail)
            def _():
                r = rpw * NW + wid
                md = m_descs(r, 0)
                _fire(md)
                _drain(md)
                sadd(0)

        plsc.subcore_barrier()

        npw = (n // NS) // 8 * 8
        nrem = n - npw * NS

        @pl.when(cid == 0)
        def _():
            pltpu.sync_copy(acc.at[pl.ds(sid * npw, npw)],
                            p0_h.at[pl.ds(sid * npw, npw)])
            if nrem:
                @pl.when(sid == 0)
                def _():
                    pltpu.sync_copy(acc.at[pl.ds(npw * NS, nrem)],
                                    p0_h.at[pl.ds(npw * NS, nrem)])

        @pl.when(cid == 1)
        def _():
            pltpu.sync_copy(acc.at[pl.ds(sid * npw, npw)],
                            p1_h.at[pl.ds(sid * npw, npw)])
            if nrem:
                @pl.when(sid == 0)
                def _():
                    pltpu.sync_copy(acc.at[pl.ds(npw * NS, nrem)],
                                    p1_h.at[pl.ds(npw * NS, nrem)])

    return body(m, dst2, zrows)


def _pad_w(w, rows, cols):
    return jnp.pad(w, ((0, rows - w.shape[0]), (0, cols - w.shape[1])))


def kernel(x_t, s_t, t, h_t, edge_index, iproj_W, iproj_b, layers,
           oc_W, oc_b, of_W, of_b):
    n = h_t.shape[0]
    e = edge_index.shape[1]
    time_dim = 16
    half = time_dim // 2

    freqs = jnp.exp(jnp.linspace(0.0, 1.0, half) * -4.0)
    angles = t[0] * freqs
    emb = jnp.concatenate([jnp.sin(angles), jnp.cos(angles)], axis=-1)
    t_emb = jnp.broadcast_to(emb[None, :], (n, time_dim))
    node_in = jnp.concatenate([h_t, s_t[:, None], t_emb], axis=-1)

    s2 = s_t[:, None]
    posfull = jnp.pad(x_t, ((0, 0), (H, F - H - 3)))
    src2 = edge_index[0].reshape(e // IB, 1, IB)
    dst2 = edge_index[1].reshape(e // IB, 1, IB)
    zrows = jnp.zeros((n, F), jnp.float32)

    pw = []
    for p in layers:
        pw.append(dict(
            wd=_pad_w(p['eW1'][:H], H, F),
            ws=_pad_w(p['eW1'][H:2 * H], H, F),
            wr=_pad_w(p['eW1'][2 * H].reshape(1, H), 1, F),
            b1e=_pad_w(p['eb1'].reshape(1, H), 1, F),
            w2p=_pad_w(p['eW2'], F, F),
            b2p=_pad_w(p['eb2'].reshape(1, H), 1, F),
            cw=_pad_w(p['cW'].reshape(1, H), 1, F),
            cb=p['cb'].reshape(1, 1),
            w1a=p['nW1'][:H],
            w1bp=_pad_w(p['nW1'][H:], F, H),
            b1=p['nb1'].reshape(1, H),
            w2=p['nW2'],
            b2=p['nb2'].reshape(1, H),
        ))

    x, dtab, stab = _proj_and_tables(
        node_in, iproj_W, iproj_b.reshape(1, H), posfull, s2,
        pw[0]['wd'], pw[0]['ws'], pw[0]['b1e'])
    pos = posfull

    for li, p in enumerate(pw):
        dg, sg = _sc_gather(dtab, stab, dst2, src2)
        m = _edge_mlp(dg, sg, p['w2p'], p['b2p'], p['cw'], p['cb'], p['wr'])
        p0, p1 = _sc_scatter(m, dst2, zrows)
        if li + 1 < len(pw):
            nxt = pw[li + 1]
            x, pos, dtab, stab = _node_update(
                p0, p1, x, pos, s2, p['w1a'], p['w1bp'], p['b1'], p['w2'],
                p['b2'], nxt['wd'], nxt['ws'], nxt['b1e'])
        else:
            x0_pred, h0_pred = _node_final(
                p0, p1, x, p['w1a'], p['w1bp'], p['b1'], p['w2'], p['b2'],
                oc_W, oc_b.reshape(1, 3), of_W,
                of_b.reshape(1, of_W.shape[1]))

    return (x0_pred, h0_pred)


# speedup vs baseline: 7.5479x; 1.0925x over previous
import functools

import jax
import jax.numpy as jnp
import numpy as np
from jax import lax
from jax.experimental import pallas as pl
from jax.experimental.pallas import tpu as pltpu
from jax.experimental.pallas import tpu_sc as plsc

H = 64
F = 128
NC = 2
NS = 16
NW = NC * NS
IB = 128
BN = 512


def _silu(v):
    return v * jax.nn.sigmoid(v)


def _masks():
    io = lax.broadcasted_iota(jnp.int32, (1, F), 1)
    auxm = jnp.where((io >= H) & (io < H + 3), 1.0, 0.0)
    oneh = jnp.where(io == H + 3, 1.0, 0.0)
    return auxm, oneh


def _edge_block(e):
    for b in (2560, 1280, 640, 512, 256, 128):
        if e % b == 0:
            return b
    return IB


def _tables(x, pos, s2, wd, ws, b1, auxsign_note=None):
    auxm, oneh = _masks()
    xd = jnp.dot(x, wd, preferred_element_type=jnp.float32) + b1
    xs = jnp.dot(x, ws, preferred_element_type=jnp.float32)
    dtab = xd + pos * auxm
    stab = xs - pos * auxm + s2 * oneh
    return dtab, stab


def _proj_body(nin_ref, w_ref, b_ref, pos_ref, s_ref, wd_ref, ws_ref, b1_ref,
               x_ref, d_ref, st_ref):
    x = _silu(
        jnp.dot(nin_ref[...], w_ref[...], preferred_element_type=jnp.float32)
        + b_ref[...])
    x_ref[...] = x
    d, st = _tables(x, pos_ref[...], s_ref[...], wd_ref[...], ws_ref[...],
                    b1_ref[...])
    d_ref[...] = d
    st_ref[...] = st


def _proj_and_tables(node_in, w, b, pos, s2, wd, ws, b1):
    n, k = node_in.shape
    grid = pl.cdiv(n, BN)
    return pl.pallas_call(
        _proj_body,
        grid=(grid,),
        in_specs=[
            pl.BlockSpec((BN, k), lambda i: (i, 0)),
            pl.BlockSpec((k, H), lambda i: (0, 0)),
            pl.BlockSpec((1, H), lambda i: (0, 0)),
            pl.BlockSpec((BN, F), lambda i: (i, 0)),
            pl.BlockSpec((BN, 1), lambda i: (i, 0)),
            pl.BlockSpec((H, F), lambda i: (0, 0)),
            pl.BlockSpec((H, F), lambda i: (0, 0)),
            pl.BlockSpec((1, F), lambda i: (0, 0)),
        ],
        out_specs=[
            pl.BlockSpec((BN, H), lambda i: (i, 0)),
            pl.BlockSpec((BN, F), lambda i: (i, 0)),
            pl.BlockSpec((BN, F), lambda i: (i, 0)),
        ],
        out_shape=[
            jax.ShapeDtypeStruct((n, H), jnp.float32),
            jax.ShapeDtypeStruct((n, F), jnp.float32),
            jax.ShapeDtypeStruct((n, F), jnp.float32),
        ],
    )(node_in, w, b, pos, s2, wd, ws, b1)


def _edge_mlp_body(ds_ref, w2_ref, b2_ref, cw_ref, cb_ref, wr_ref,
                   o_ref):
    auxm, oneh = _masks()
    ds = ds_ref[...]
    aux = ds * auxm
    r2 = jnp.sum(aux * aux, axis=1, keepdims=True)
    h1 = _silu(ds + r2 * wr_ref[...])
    m = _silu(
        jnp.dot(h1, w2_ref[...], preferred_element_type=jnp.float32)
        + b2_ref[...])
    scol = jnp.sum(ds * oneh, axis=1, keepdims=True)
    m = m * scol
    gamma = jnp.sum(m * cw_ref[...], axis=1, keepdims=True) + cb_ref[...]
    o_ref[...] = m + gamma * aux + oneh


def _edge_mlp(ds, w2p, b2p, cw, cb, wr):
    e = ds.shape[0]
    be = _edge_block(e)
    grid = e // be
    return pl.pallas_call(
        _edge_mlp_body,
        grid=(grid,),
        in_specs=[
            pl.BlockSpec((be, F), lambda i: (i, 0)),
            pl.BlockSpec((F, F), lambda i: (0, 0)),
            pl.BlockSpec((1, F), lambda i: (0, 0)),
            pl.BlockSpec((1, F), lambda i: (0, 0)),
            pl.BlockSpec((1, 1), lambda i: (0, 0)),
            pl.BlockSpec((1, F), lambda i: (0, 0)),
        ],
        out_specs=pl.BlockSpec((be, F), lambda i: (i, 0)),
        out_shape=jax.ShapeDtypeStruct((e, F), jnp.float32),
    )(ds, w2p, b2p, cw, cb, wr)


def _node_core(p0, p1, x, w1a, w1bp, b1, w2, b2):
    auxm, oneh = _masks()
    acc = p0 + p1
    deg = jnp.sum(acc * oneh, axis=1, keepdims=True)
    degc = jnp.maximum(deg, 1.0)
    acn = acc / degc
    h = _silu(
        jnp.dot(x, w1a, preferred_element_type=jnp.float32)
        + jnp.dot(acn, w1bp, preferred_element_type=jnp.float32)
        + b1)
    xn = jnp.dot(h, w2, preferred_element_type=jnp.float32) + b2
    return xn, acc * auxm / degc


def _nu_body(p0_ref, p1_ref, x_ref, pos_ref, s_ref, w1a_ref, w1b_ref, b1_ref,
             w2_ref, b2_ref, wd_ref, ws_ref, b1e_ref,
             xo_ref, poso_ref, d_ref, st_ref):
    auxm, _ = _masks()
    xn, cupd = _node_core(p0_ref[...], p1_ref[...], x_ref[...], w1a_ref[...],
                          w1b_ref[...], b1_ref[...], w2_ref[...], b2_ref[...])
    posn = pos_ref[...] + cupd
    xo_ref[...] = xn
    poso_ref[...] = posn
    d, st = _tables(xn, posn, s_ref[...], wd_ref[...], ws_ref[...],
                    b1e_ref[...])
    d_ref[...] = d
    st_ref[...] = st


def _node_update(p0, p1, x, pos, s2, w1a, w1bp, b1, w2, b2, wd, ws, b1e):
    n = x.shape[0]
    grid = pl.cdiv(n, BN)
    return pl.pallas_call(
        _nu_body,
        grid=(grid,),
        in_specs=[
            pl.BlockSpec((BN, F), lambda i: (i, 0)),
            pl.BlockSpec((BN, F), lambda i: (i, 0)),
            pl.BlockSpec((BN, H), lambda i: (i, 0)),
            pl.BlockSpec((BN, F), lambda i: (i, 0)),
            pl.BlockSpec((BN, 1), lambda i: (i, 0)),
            pl.BlockSpec((H, H), lambda i: (0, 0)),
            pl.BlockSpec((F, H), lambda i: (0, 0)),
            pl.BlockSpec((1, H), lambda i: (0, 0)),
            pl.BlockSpec((H, H), lambda i: (0, 0)),
            pl.BlockSpec((1, H), lambda i: (0, 0)),
            pl.BlockSpec((H, F), lambda i: (0, 0)),
            pl.BlockSpec((H, F), lambda i: (0, 0)),
            pl.BlockSpec((1, F), lambda i: (0, 0)),
        ],
        out_specs=[
            pl.BlockSpec((BN, H), lambda i: (i, 0)),
            pl.BlockSpec((BN, F), lambda i: (i, 0)),
            pl.BlockSpec((BN, F), lambda i: (i, 0)),
            pl.BlockSpec((BN, F), lambda i: (i, 0)),
        ],
        out_shape=[
            jax.ShapeDtypeStruct((n, H), jnp.float32),
            jax.ShapeDtypeStruct((n, F), jnp.float32),
            jax.ShapeDtypeStruct((n, F), jnp.float32),
            jax.ShapeDtypeStruct((n, F), jnp.float32),
        ],
    )(p0, p1, x, pos, s2, w1a, w1bp, b1, w2, b2, wd, ws, b1e)


def _final_body(p0_ref, p1_ref, x_ref, w1a_ref, w1b_ref, b1_ref, w2_ref,
                b2_ref, wc_ref, bc_ref, wf_ref, bf_ref, xo_ref, ho_ref):
    xn, _ = _node_core(p0_ref[...], p1_ref[...], x_ref[...], w1a_ref[...],
                       w1b_ref[...], b1_ref[...], w2_ref[...], b2_ref[...])
    xo_ref[...] = (
        jnp.dot(xn, wc_ref[...], preferred_element_type=jnp.float32)
        + bc_ref[...])
    ho_ref[...] = (
        jnp.dot(xn, wf_ref[...], preferred_element_type=jnp.float32)
        + bf_ref[...])


def _node_final(p0, p1, x, w1a, w1bp, b1, w2, b2, oc_w, oc_b, of_w, of_b):
    n = x.shape[0]
    nd = of_w.shape[1]
    grid = pl.cdiv(n, BN)
    return pl.pallas_call(
        _final_body,
        grid=(grid,),
        in_specs=[
            pl.BlockSpec((BN, F), lambda i: (i, 0)),
            pl.BlockSpec((BN, F), lambda i: (i, 0)),
            pl.BlockSpec((BN, H), lambda i: (i, 0)),
            pl.BlockSpec((H, H), lambda i: (0, 0)),
            pl.BlockSpec((F, H), lambda i: (0, 0)),
            pl.BlockSpec((1, H), lambda i: (0, 0)),
            pl.BlockSpec((H, H), lambda i: (0, 0)),
            pl.BlockSpec((1, H), lambda i: (0, 0)),
            pl.BlockSpec((H, 3), lambda i: (0, 0)),
            pl.BlockSpec((1, 3), lambda i: (0, 0)),
            pl.BlockSpec((H, nd), lambda i: (0, 0)),
            pl.BlockSpec((1, nd), lambda i: (0, 0)),
        ],
        out_specs=[
            pl.BlockSpec((BN, 3), lambda i: (i, 0)),
            pl.BlockSpec((BN, nd), lambda i: (i, 0)),
        ],
        out_shape=[
            jax.ShapeDtypeStruct((n, 3), jnp.float32),
            jax.ShapeDtypeStruct((n, nd), jnp.float32),
        ],
    )(p0, p1, x, w1a, w1bp, b1, w2, b2, oc_w, oc_b, of_w, of_b)


def _sc_mesh():
    return plsc.VectorSubcoreMesh(
        core_axis_name="c", subcore_axis_name="s",
        num_cores=NC, num_subcores=NS)


def _fire(descs):
    for d in descs:
        d.start()


def _drain(descs):
    for d in descs:
        d.wait()


def _sc_gather(dtab, stab, dst2, src2):
    rows = dst2.shape[0]
    e = rows * IB
    rpw = rows // NW
    tail = rows - rpw * NW
    assert rpw % 2 == 0
    ng2 = rpw // 2

    @functools.partial(
        pl.kernel,
        out_type=jax.ShapeDtypeStruct((e, F), jnp.float32),
        mesh=_sc_mesh(),
        scratch_types=[
            pltpu.VMEM((2, 1, IB), jnp.int32),
            pltpu.VMEM((2, 1, IB), jnp.int32),
            pltpu.VMEM((2 * IB, F), jnp.float32),
            pltpu.VMEM((2 * IB, F), jnp.float32),
            pltpu.SemaphoreType.DMA,
            pltpu.SemaphoreType.DMA,
            pltpu.SemaphoreType.DMA,
            pltpu.SemaphoreType.DMA,
        ],
    )
    def body(dtab_h, stab_h, dst_h, src_h, ds_h,
             idxd, idxs, dbuf, sbuf, gsem0, gsem1, wsem0, wsem1):
        wid = lax.axis_index("s") * NC + lax.axis_index("c")
        row0 = wid * rpw
        gsems = (gsem0, gsem1)
        wsems = (wsem0, wsem1)

        def load_idx(r, slot):
            pltpu.sync_copy(dst_h.at[pl.ds(r, 1)], idxd.at[pl.ds(slot, 1)])
            pltpu.sync_copy(src_h.at[pl.ds(r, 1)], idxs.at[pl.ds(slot, 1)])

        def g_descs(slot):
            return (
                pltpu.make_async_copy(dtab_h.at[idxd.at[slot, 0]],
                                      dbuf.at[pl.ds(slot * IB, IB)],
                                      gsems[slot]),
                pltpu.make_async_copy(stab_h.at[idxs.at[slot, 0]],
                                      sbuf.at[pl.ds(slot * IB, IB)],
                                      gsems[slot]),
            )

        def w_descs(r, slot):
            return (
                pltpu.make_async_copy(dbuf.at[pl.ds(slot * IB, IB)],
                                      ds_h.at[pl.ds(r * IB, IB)],
                                      wsems[slot]),
            )

        def add_rows(slot):
            def row_step(i, carry):
                row = slot * IB + i
                for j in range(F // 16):
                    lanes = pl.ds(j * 16, 16)
                    dbuf[row, lanes] = dbuf[row, lanes] + sbuf[row, lanes]
                return carry
            lax.fori_loop(0, IB, row_step, 0)

        load_idx(row0, 0)
        _fire(g_descs(0))

        def pair(g2, carry):
            r = row0 + 2 * g2

            @pl.when(g2 >= 1)
            def _():
                _drain(w_descs(r, 1))
            load_idx(r + 1, 1)
            _fire(g_descs(1))
            _drain(g_descs(0))
            add_rows(0)
            _fire(w_descs(r, 0))

            _drain(w_descs(r, 0))
            @pl.when(g2 < ng2 - 1)
            def _():
                load_idx(r + 2, 0)
                _fire(g_descs(0))
            _drain(g_descs(1))
            add_rows(1)
            _fire(w_descs(r + 1, 1))
            return carry

        lax.fori_loop(0, ng2, pair, 0)
        _drain(w_descs(row0, 1))

        if tail:
            @pl.when(wid < tail)
            def _():
                r = rpw * NW + wid
                load_idx(r, 0)
                gd = g_descs(0)
                _fire(gd)
                _drain(gd)
                add_rows(0)
                wd = w_descs(r, 0)
                _fire(wd)
                _drain(wd)

    return body(dtab, stab, dst2, src2)


def _sc_scatter(m, dst2, zrows):
    n = zrows.shape[0]
    rows = dst2.shape[0]
    rpw = rows // NW
    tail = rows - rpw * NW
    assert rpw % 2 == 0
    ng2 = rpw // 2

    @functools.partial(
        pl.kernel,
        out_type=[
            jax.ShapeDtypeStruct((n, F), jnp.float32),
            jax.ShapeDtypeStruct((n, F), jnp.float32),
        ],
        mesh=_sc_mesh(),
        scratch_types=[
            pltpu.VMEM((2, 1, IB), jnp.int32),
            pltpu.VMEM((2 * IB, F), jnp.float32),
            pltpu.VMEM_SHARED((n, F), jnp.float32),
            pltpu.SemaphoreType.DMA,
            pltpu.SemaphoreType.DMA,
        ],
    )
    def body(m_h, dst_h, z_h, p0_h, p1_h, idxd, mbuf, acc, msem0, msem1):
        cid = lax.axis_index("c")
        sid = lax.axis_index("s")
        wid = sid * NC + cid
        row0 = wid * rpw
        msems = (msem0, msem1)

        @pl.when(sid == 0)
        def _():
            pltpu.sync_copy(z_h, acc)

        plsc.subcore_barrier()

        def m_descs(r, slot):
            return (
                pltpu.make_async_copy(m_h.at[pl.ds(r * IB, IB)],
                                      mbuf.at[pl.ds(slot * IB, IB)],
                                      msems[slot]),
                pltpu.make_async_copy(dst_h.at[pl.ds(r, 1)],
                                      idxd.at[pl.ds(slot, 1)],
                                      msems[slot]),
            )

        def sadd(slot):
            pltpu.sync_copy(mbuf.at[pl.ds(slot * IB, IB)],
                            acc.at[idxd.at[slot, 0]], add=True)

        _fire(m_descs(row0, 0))

        def pair(g2, carry):
            r = row0 + 2 * g2
            _fire(m_descs(r + 1, 1))
            _drain(m_descs(r, 0))
            sadd(0)
            @pl.when(g2 < ng2 - 1)
            def _():
                _fire(m_descs(r + 2, 0))
            _drain(m_descs(r + 1, 1))
            sadd(1)
            return carry

        lax.fori_loop(0, ng2, pair, 0)

        if tail:
            @pl.when(wid < tail)
            def _():
                r = rpw * NW + wid
                md = m_descs(r, 0)
                _fire(md)
                _drain(md)
                sadd(0)

        plsc.subcore_barrier()

        npw = (n // NS) // 8 * 8
        nrem = n - npw * NS

        @pl.when(cid == 0)
        def _():
            pltpu.sync_copy(acc.at[pl.ds(sid * npw, npw)],
                            p0_h.at[pl.ds(sid * npw, npw)])
            if nrem:
                @pl.when(sid == 0)
                def _():
                    pltpu.sync_copy(acc.at[pl.ds(npw * NS, nrem)],
                                    p0_h.at[pl.ds(npw * NS, nrem)])

        @pl.when(cid == 1)
        def _():
            pltpu.sync_copy(acc.at[pl.ds(sid * npw, npw)],
                            p1_h.at[pl.ds(sid * npw, npw)])
            if nrem:
                @pl.when(sid == 0)
                def _():
                    pltpu.sync_copy(acc.at[pl.ds(npw * NS, nrem)],
                                    p1_h.at[pl.ds(npw * NS, nrem)])

    return body(m, dst2, zrows)


def _pad_w(w, rows, cols):
    return jnp.pad(w, ((0, rows - w.shape[0]), (0, cols - w.shape[1])))


def kernel(x_t, s_t, t, h_t, edge_index, iproj_W, iproj_b, layers,
           oc_W, oc_b, of_W, of_b):
    n = h_t.shape[0]
    e = edge_index.shape[1]
    time_dim = 16
    half = time_dim // 2

    freqs = jnp.exp(jnp.linspace(0.0, 1.0, half) * -4.0)
    angles = t[0] * freqs
    emb = jnp.concatenate([jnp.sin(angles), jnp.cos(angles)], axis=-1)
    t_emb = jnp.broadcast_to(emb[None, :], (n, time_dim))
    node_in = jnp.concatenate([h_t, s_t[:, None], t_emb], axis=-1)

    s2 = s_t[:, None]
    posfull = jnp.pad(x_t, ((0, 0), (H, F - H - 3)))
    src2 = edge_index[0].reshape(e // IB, 1, IB)
    dst2 = edge_index[1].reshape(e // IB, 1, IB)
    zrows = jnp.zeros((n, F), jnp.float32)

    pw = []
    for p in layers:
        pw.append(dict(
            wd=_pad_w(p['eW1'][:H], H, F),
            ws=_pad_w(p['eW1'][H:2 * H], H, F),
            wr=_pad_w(p['eW1'][2 * H].reshape(1, H), 1, F),
            b1e=_pad_w(p['eb1'].reshape(1, H), 1, F),
            w2p=_pad_w(p['eW2'], F, F),
            b2p=_pad_w(p['eb2'].reshape(1, H), 1, F),
            cw=_pad_w(p['cW'].reshape(1, H), 1, F),
            cb=p['cb'].reshape(1, 1),
            w1a=p['nW1'][:H],
            w1bp=_pad_w(p['nW1'][H:], F, H),
            b1=p['nb1'].reshape(1, H),
            w2=p['nW2'],
            b2=p['nb2'].reshape(1, H),
        ))

    x, dtab, stab = _proj_and_tables(
        node_in, iproj_W, iproj_b.reshape(1, H), posfull, s2,
        pw[0]['wd'], pw[0]['ws'], pw[0]['b1e'])
    pos = posfull

    for li, p in enumerate(pw):
        ds = _sc_gather(dtab, stab, dst2, src2)
        m = _edge_mlp(ds, p['w2p'], p['b2p'], p['cw'], p['cb'], p['wr'])
        p0, p1 = _sc_scatter(m, dst2, zrows)
        if li + 1 < len(pw):
            nxt = pw[li + 1]
            x, pos, dtab, stab = _node_update(
                p0, p1, x, pos, s2, p['w1a'], p['w1bp'], p['b1'], p['w2'],
                p['b2'], nxt['wd'], nxt['ws'], nxt['b1e'])
        else:
            x0_pred, h0_pred = _node_final(
                p0, p1, x, p['w1a'], p['w1bp'], p['b1'], p['w2'], p['b2'],
                oc_W, oc_b.reshape(1, 3), of_W,
                of_b.reshape(1, of_W.shape[1]))

    return (x0_pred, h0_pred)
